# Initial kernel scaffold; baseline (speedup 1.0000x reference)
import jax
import jax.numpy as jnp
from jax.experimental import pallas as pl


def kernel(continuous, categorical_indices, tables, W0, b0, W1, b1, W2, b2):
    raise NotImplementedError("write your pallas kernel here")


# speedup vs baseline: 7.1134x; 7.1134x over previous
import functools

import jax
import jax.numpy as jnp
from jax import lax
from jax.experimental import pallas as pl
from jax.experimental.pallas import tpu as pltpu
from jax.experimental.pallas import tpu_sc as plsc

B = 16384
F = 26
V = 100000
D = 32
C = 16
H0, H1, H2 = 512, 256, 128

NC, NS = 2, 16
NW = NC * NS
TOTAL = B * F
CHUNK = 128
N_CHUNKS = TOTAL // CHUNK
CPW = N_CHUNKS // NW

_mesh = plsc.VectorSubcoreMesh(core_axis_name="c", subcore_axis_name="s")


@functools.partial(
    pl.kernel,
    out_type=jax.ShapeDtypeStruct((TOTAL, D), jnp.float32),
    mesh=_mesh,
    scratch_types=[
        pltpu.VMEM((CPW, CHUNK), jnp.int32),
        pltpu.VMEM((CHUNK, D), jnp.float32),
        pltpu.SemaphoreType.DMA,
    ],
    compiler_params=pltpu.CompilerParams(use_tc_tiling_on_sc=False),
)
def _sc_gather(idx_hbm, tables_hbm, out_hbm, idx_v, rows_v, sem):
    wid = lax.axis_index("s") * NC + lax.axis_index("c")
    c0 = wid * CPW
    pltpu.sync_copy(idx_hbm.at[pl.ds(c0, CPW), :], idx_v)

    def body(j, carry):
        pltpu.async_copy(tables_hbm.at[idx_v.at[j]], rows_v, sem).wait()
        pltpu.sync_copy(rows_v, out_hbm.at[pl.ds((c0 + j) * CHUNK, CHUNK), :])
        return carry

    lax.fori_loop(0, CPW, body, 0)


BM = 512


def _mlp_body(emb_ref, cont_ref, w0a_ref, w0b_ref, b0_ref, w1_ref, b1_ref,
              w2_ref, b2_ref, out_ref):
    h0 = jnp.dot(emb_ref[...], w0a_ref[...], preferred_element_type=jnp.float32)
    h0 += jnp.dot(cont_ref[...], w0b_ref[...], preferred_element_type=jnp.float32)
    h0 = jnp.maximum(h0 + b0_ref[...], 0.0)
    h1 = jnp.maximum(
        jnp.dot(h0, w1_ref[...], preferred_element_type=jnp.float32) + b1_ref[...], 0.0)
    out_ref[...] = jnp.maximum(
        jnp.dot(h1, w2_ref[...], preferred_element_type=jnp.float32) + b2_ref[...], 0.0)


_mlp = pl.pallas_call(
    _mlp_body,
    grid=(B // BM,),
    in_specs=[
        pl.BlockSpec((BM, F * D), lambda i: (i, 0)),
        pl.BlockSpec((BM, C), lambda i: (i, 0)),
        pl.BlockSpec((F * D, H0), lambda i: (0, 0)),
        pl.BlockSpec((C, H0), lambda i: (0, 0)),
        pl.BlockSpec((1, H0), lambda i: (0, 0)),
        pl.BlockSpec((H0, H1), lambda i: (0, 0)),
        pl.BlockSpec((1, H1), lambda i: (0, 0)),
        pl.BlockSpec((H1, H2), lambda i: (0, 0)),
        pl.BlockSpec((1, H2), lambda i: (0, 0)),
    ],
    out_specs=pl.BlockSpec((BM, H2), lambda i: (i, 0)),
    out_shape=jax.ShapeDtypeStruct((B, H2), jnp.float32),
)


def kernel(continuous, categorical_indices, tables, W0, b0, W1, b1, W2, b2):
    offsets = (jnp.arange(F, dtype=jnp.int32) * V)[None, :]
    flat_idx = (categorical_indices + offsets).reshape(N_CHUNKS, CHUNK)
    emb_flat = _sc_gather(flat_idx, tables)
    emb = emb_flat.reshape(B, F * D)
    return _mlp(emb, continuous.astype(jnp.float32),
                W0[:F * D], W0[F * D:],
                b0.reshape(1, H0), W1, b1.reshape(1, H1),
                W2, b2.reshape(1, H2))
